# baseline (speedup 1.0000x reference)
import functools

import jax
import jax.numpy as jnp
from jax import lax
from jax.experimental import pallas as pl
from jax.experimental.pallas import tpu as pltpu
from jax.experimental.pallas import tpu_sc as plsc

_NC = 2
_NS = 16
_NW = _NC * _NS
_L = 16


@functools.lru_cache(maxsize=None)
def _make_sc_kernel(B, K):
    assert B % (_NW * _L) == 0
    assert K % _L == 0 and (K & (K - 1)) == 0
    bpw = B // _NW

    mesh = plsc.VectorSubcoreMesh(core_axis_name="c", subcore_axis_name="s")

    @functools.partial(
        pl.kernel,
        mesh=mesh,
        compiler_params=pltpu.CompilerParams(
            use_tc_tiling_on_sc=False,
            needs_layout_passes=False,
        ),
        out_type=(
            jax.ShapeDtypeStruct((B,), jnp.float32),
            jax.ShapeDtypeStruct((B,), jnp.float32),
        ),
        scratch_types=[
            pltpu.VMEM((bpw,), jnp.int32),
            pltpu.VMEM((bpw,), jnp.int32),
            pltpu.VMEM((bpw,), jnp.int32),
            pltpu.VMEM((bpw, K), jnp.float32),
            pltpu.VMEM((bpw, K), jnp.float32),
            pltpu.VMEM((bpw, K), jnp.float32),
            pltpu.VMEM((bpw,), jnp.float32),
            pltpu.VMEM((bpw,), jnp.float32),
            pltpu.VMEM((bpw,), jnp.float32),
            pltpu.VMEM((bpw,), jnp.float32),
            pltpu.VMEM((bpw,), jnp.float32),
            pltpu.SemaphoreType.DMA,
        ],
    )
    def sc_kernel(u_hbm, i_hbm, j_hbm, gu_hbm, gi_hbm, bu_hbm, bi_hbm,
                  xui_hbm, xuj_hbm,
                  u_idx, i_idx, j_idx, u_rows, i_rows, j_rows,
                  bu_v, bi_v, bj_v, xui_v, xuj_v, sem):
        wid = lax.axis_index("s") * _NC + lax.axis_index("c")
        base = wid * bpw

        pltpu.sync_copy(u_hbm.at[pl.ds(base, bpw)], u_idx)
        pltpu.sync_copy(i_hbm.at[pl.ds(base, bpw)], i_idx)
        pltpu.sync_copy(j_hbm.at[pl.ds(base, bpw)], j_idx)

        copies = [
            pltpu.async_copy(gu_hbm.at[u_idx], u_rows, sem),
            pltpu.async_copy(gi_hbm.at[i_idx], i_rows, sem),
            pltpu.async_copy(gi_hbm.at[j_idx], j_rows, sem),
            pltpu.async_copy(bu_hbm.at[u_idx], bu_v, sem),
            pltpu.async_copy(bi_hbm.at[i_idx], bi_v, sem),
            pltpu.async_copy(bi_hbm.at[j_idx], bj_v, sem),
        ]
        for c in copies:
            c.wait()

        lane = lax.iota(jnp.int32, _L)

        def blk_body(blk, carry):
            off = blk * _L
            rids = off + lane
            bu = bu_v[pl.ds(off, _L)]
            acc_ui = bu + bi_v[pl.ds(off, _L)]
            acc_uj = bu + bj_v[pl.ds(off, _L)]
            for k in range(K):
                ck = lax.bitwise_and(lane + k, K - 1)
                vu = plsc.load_gather(u_rows, [rids, ck])
                vi = plsc.load_gather(i_rows, [rids, ck])
                vj = plsc.load_gather(j_rows, [rids, ck])
                acc_ui = acc_ui + vu * vi
                acc_uj = acc_uj + vu * vj
            xui_v[pl.ds(off, _L)] = acc_ui
            xuj_v[pl.ds(off, _L)] = acc_uj
            return carry

        lax.fori_loop(0, bpw // _L, blk_body, 0)

        pltpu.sync_copy(xui_v, xui_hbm.at[pl.ds(base, bpw)])
        pltpu.sync_copy(xuj_v, xuj_hbm.at[pl.ds(base, bpw)])

    return sc_kernel


def _loss_body(xui_ref, xuj_ref, out_ref):
    d = xui_ref[...] - xuj_ref[...]
    ls = jnp.minimum(d, 0.0) - jnp.log1p(jnp.exp(-jnp.abs(d)))
    out_ref[0, 0] = -jnp.sum(ls) / d.size


@functools.lru_cache(maxsize=None)
def _make_loss_kernel(B):
    return pl.pallas_call(
        _loss_body,
        out_shape=jax.ShapeDtypeStruct((1, 1), jnp.float32),
        out_specs=pl.BlockSpec(memory_space=pltpu.SMEM),
    )


def kernel(u, i, j, kps, gamma_u, gamma_i, beta_u, beta_i):
    B = u.shape[0]
    K = gamma_u.shape[1]
    sc = _make_sc_kernel(B, K)
    x_ui, x_uj = sc(
        u.astype(jnp.int32), i.astype(jnp.int32), j.astype(jnp.int32),
        gamma_u, gamma_i,
        beta_u.reshape(-1), beta_i.reshape(-1),
    )
    loss2d = _make_loss_kernel(B)(
        x_ui.reshape(B // 128, 128), x_uj.reshape(B // 128, 128)
    )
    return x_ui, x_uj, loss2d[0, 0]


# speedup vs baseline: 1.0072x; 1.0072x over previous
import functools

import jax
import jax.numpy as jnp
from jax import lax
from jax.experimental import pallas as pl
from jax.experimental.pallas import tpu as pltpu
from jax.experimental.pallas import tpu_sc as plsc

_NC = 2
_NS = 16
_NW = _NC * _NS
_L = 16
_CH = 256


@functools.lru_cache(maxsize=None)
def _make_sc_kernel(B, K):
    assert B % (_NW * _CH) == 0
    assert K == 64
    bpw = B // _NW
    nchunk = bpw // _CH
    nblk = _CH // _L

    mesh = plsc.VectorSubcoreMesh(core_axis_name="c", subcore_axis_name="s")

    @functools.partial(
        pl.kernel,
        mesh=mesh,
        compiler_params=pltpu.CompilerParams(needs_layout_passes=False),
        out_type=(
            jax.ShapeDtypeStruct((B,), jnp.float32),
            jax.ShapeDtypeStruct((B,), jnp.float32),
        ),
        scratch_types=[
            pltpu.VMEM((bpw,), jnp.int32),
            pltpu.VMEM((bpw,), jnp.int32),
            pltpu.VMEM((bpw,), jnp.int32),
            pltpu.VMEM((_CH,), jnp.int32),
            pltpu.VMEM((_CH,), jnp.int32),
            pltpu.VMEM((_CH,), jnp.int32),
            pltpu.VMEM((_CH, 2 * K), jnp.float32),
            pltpu.VMEM((_CH, 2 * K), jnp.float32),
            pltpu.VMEM((_CH, 2 * K), jnp.float32),
            pltpu.VMEM((bpw,), jnp.float32),
            pltpu.VMEM((bpw,), jnp.float32),
            pltpu.VMEM((bpw,), jnp.float32),
            pltpu.VMEM((bpw,), jnp.float32),
            pltpu.VMEM((bpw,), jnp.float32),
            pltpu.SemaphoreType.DMA,
        ],
    )
    def sc_kernel(u_hbm, i_hbm, j_hbm, gu_hbm, gi_hbm, bu_hbm, bi_hbm,
                  xui_hbm, xuj_hbm,
                  u_idx, i_idx, j_idx, nu, ni, nj, u_rows, i_rows, j_rows,
                  bu_v, bi_v, bj_v, xui_v, xuj_v, sem):
        wid = lax.axis_index("s") * _NC + lax.axis_index("c")
        base = wid * bpw

        pltpu.sync_copy(u_hbm.at[pl.ds(base, bpw)], u_idx)
        pltpu.sync_copy(i_hbm.at[pl.ds(base, bpw)], i_idx)
        pltpu.sync_copy(j_hbm.at[pl.ds(base, bpw)], j_idx)

        beta_copies = [
            pltpu.async_copy(bu_hbm.at[u_idx], bu_v, sem),
            pltpu.async_copy(bi_hbm.at[i_idx], bi_v, sem),
            pltpu.async_copy(bi_hbm.at[j_idx], bj_v, sem),
        ]

        lane = lax.iota(jnp.int32, _L)

        for c in range(nchunk):
            coff = c * _CH
            for b in range(nblk):
                s_src = pl.ds(coff + b * _L, _L)
                s_dst = pl.ds(b * _L, _L)
                nu[s_dst] = lax.shift_right_logical(u_idx[s_src], 1)
                ni[s_dst] = lax.shift_right_logical(i_idx[s_src], 1)
                nj[s_dst] = lax.shift_right_logical(j_idx[s_src], 1)
            row_copies = [
                pltpu.async_copy(gu_hbm.at[nu], u_rows, sem),
                pltpu.async_copy(gi_hbm.at[ni], i_rows, sem),
                pltpu.async_copy(gi_hbm.at[nj], j_rows, sem),
            ]
            if c == 0:
                for bc in beta_copies:
                    bc.wait()
            for rc in row_copies:
                rc.wait()

            def blk_body(blk, carry):
                off = coff + blk * _L
                sl = pl.ds(off, _L)
                rids = blk * _L + lane
                pu = lax.bitwise_and(u_idx[sl], 1) * K
                pi = lax.bitwise_and(i_idx[sl], 1) * K
                pj = lax.bitwise_and(j_idx[sl], 1) * K
                bu = bu_v[sl]
                acc_ui = bu + bi_v[sl]
                acc_uj = bu + bj_v[sl]
                for k in range(K):
                    ck = lax.bitwise_and(lane + k, K - 1)
                    vu = plsc.load_gather(u_rows, [rids, pu + ck])
                    vi = plsc.load_gather(i_rows, [rids, pi + ck])
                    vj = plsc.load_gather(j_rows, [rids, pj + ck])
                    acc_ui = acc_ui + vu * vi
                    acc_uj = acc_uj + vu * vj
                xui_v[sl] = acc_ui
                xuj_v[sl] = acc_uj
                return carry

            lax.fori_loop(0, nblk, blk_body, 0)

        pltpu.sync_copy(xui_v, xui_hbm.at[pl.ds(base, bpw)])
        pltpu.sync_copy(xuj_v, xuj_hbm.at[pl.ds(base, bpw)])

    return sc_kernel


def _loss_body(xui_ref, xuj_ref, out_ref):
    d = xui_ref[...] - xuj_ref[...]
    ls = jnp.minimum(d, 0.0) - jnp.log1p(jnp.exp(-jnp.abs(d)))
    out_ref[0, 0] = -jnp.sum(ls) / d.size


@functools.lru_cache(maxsize=None)
def _make_loss_kernel(B):
    return pl.pallas_call(
        _loss_body,
        out_shape=jax.ShapeDtypeStruct((1, 1), jnp.float32),
        out_specs=pl.BlockSpec(memory_space=pltpu.SMEM),
    )


def kernel(u, i, j, kps, gamma_u, gamma_i, beta_u, beta_i):
    B = u.shape[0]
    K = gamma_u.shape[1]
    sc = _make_sc_kernel(B, K)
    x_ui, x_uj = sc(
        u.astype(jnp.int32), i.astype(jnp.int32), j.astype(jnp.int32),
        gamma_u.reshape(-1, 2 * K), gamma_i.reshape(-1, 2 * K),
        beta_u.reshape(-1), beta_i.reshape(-1),
    )
    loss2d = _make_loss_kernel(B)(
        x_ui.reshape(B // 128, 128), x_uj.reshape(B // 128, 128)
    )
    return x_ui, x_uj, loss2d[0, 0]
